# baseline (speedup 1.0000x reference)
import jax
import jax.numpy as jnp
from jax import lax
from jax.experimental import pallas as pl
from jax.experimental.pallas import tpu as pltpu
from jax.experimental.pallas import tpu_sc as plsc

NUM_NEWS = 100000
NUM_USERS = 1000000
MAX_LEN = 20
D = 64
B = 4096
HIST = 50

NC = 2
NS = 16
NW = NC * NS
L = 16

_SC_PARAMS = pltpu.CompilerParams(needs_layout_passes=False,
                                  use_tc_tiling_on_sc=False)


C1 = 16
NT1 = C1 * MAX_LEN
NCHUNK1 = NUM_NEWS // C1
JMAX1 = (NCHUNK1 + NW - 1) // NW
ISL = 128
SL1 = ((0, 128), (128, 128), (256, 64))
NBUF1 = 4


def _news_pool_body(text_hbm, table_hbm, out_hbm,
                    idx0, idx1, idx2, idx3, rows0, rows1, rows2, rows3,
                    out_v, isem0, isem1, isem2, isem3,
                    gsem0, gsem1, gsem2, gsem3):
  wid = lax.axis_index("s") * NC + lax.axis_index("c")
  lanes = lax.iota(jnp.int32, L)
  bufs = ((idx0, rows0, isem0, gsem0), (idx1, rows1, isem1, gsem1),
          (idx2, rows2, isem2, gsem2), (idx3, rows3, isem3, gsem3))

  def g_of(j):
    return j * NW + wid

  def idx_copy(j, p):
    idx, _, isem, _ = bufs[p]
    g = g_of(j)

    @pl.when(g < NCHUNK1)
    def _():
      pltpu.make_async_copy(text_hbm.at[pl.ds(g * NT1, NT1)], idx,
                            isem).start()

  def idx_wait(j, p):
    idx, _, isem, _ = bufs[p]

    @pl.when(g_of(j) < NCHUNK1)
    def _():
      pltpu.make_async_copy(text_hbm.at[pl.ds(0, NT1)], idx, isem).wait()

  def gather_start(j, p):
    idx, rows, _, gsem = bufs[p]

    @pl.when(g_of(j) < NCHUNK1)
    def _():
      for o, n in SL1:
        pltpu.make_async_copy(
            table_hbm.at[idx.at[pl.ds(o, n)]],
            rows.at[pl.ds(o, n)], gsem).start()

  def gather_wait(j, p):
    idx, rows, _, gsem = bufs[p]

    @pl.when(g_of(j) < NCHUNK1)
    def _():
      for o, n in SL1:
        pltpu.make_async_copy(
            table_hbm.at[idx.at[pl.ds(o, n)]],
            rows.at[pl.ds(o, n)], gsem).wait()

  def compute(j, p):
    idx, rows, _, _ = bufs[p]
    g = g_of(j)

    @pl.when(g < NCHUNK1)
    def _():
      def row_body(c, _):
        base = c * MAX_LEN
        a = idx[pl.ds(base, L)]
        b = idx[pl.ds(base + 4, L)]
        ones = jnp.ones((L,), jnp.float32)
        zeros = jnp.zeros((L,), jnp.float32)
        cnt = (jnp.sum(jnp.where(a != 0, ones, zeros)) +
               jnp.sum(jnp.where((b != 0) & (lanes >= 12), ones, zeros)))
        s = ones / jnp.maximum(jnp.full((L,), cnt), 1e-9)
        mask_hi = jnp.full((L,), 0xFFFF0000, jnp.uint32)
        for h in range(D // 32):
          acc_lo = jnp.zeros((L,), jnp.float32)
          acc_hi = jnp.zeros((L,), jnp.float32)
          for t in range(MAX_LEN):
            w = plsc.bitcast(rows[base + t, pl.ds(h * 32, 32)], jnp.uint32)
            acc_lo = acc_lo + plsc.bitcast(w << 16, jnp.float32)
            acc_hi = acc_hi + plsc.bitcast(w & mask_hi, jnp.float32)
          cvec = jnp.full((L,), c, jnp.int32)
          plsc.store_scatter(out_v, [cvec, h * 32 + 2 * lanes], acc_lo * s)
          plsc.store_scatter(out_v, [cvec, h * 32 + 2 * lanes + 1],
                             acc_hi * s)
        return 0

      lax.fori_loop(0, C1, row_body, 0)
      pltpu.sync_copy(out_v, out_hbm.at[pl.ds(g * C1, C1)])

  for p in range(NBUF1):
    idx_copy(p, p)
  idx_wait(0, 0)
  gather_start(0, 0)
  idx_wait(1, 1)
  gather_start(1, 1)

  def loop_body(jj, _):
    for p in range(NBUF1):
      j = NBUF1 * jj + p
      p2 = (p + 2) % NBUF1
      idx_wait(j + 2, p2)
      gather_start(j + 2, p2)
      gather_wait(j, p)
      compute(j, p)
      idx_copy(j + NBUF1, p)
    return 0

  lax.fori_loop(0, JMAX1 // NBUF1, loop_body, 0)


@jax.jit
def _news_pool(news_text_flat, emb_table):
  mesh = plsc.VectorSubcoreMesh(core_axis_name="c", subcore_axis_name="s")
  kern = pl.kernel(
      _news_pool_body,
      out_type=jax.ShapeDtypeStruct((NUM_NEWS, D), jnp.float32),
      mesh=mesh,
      compiler_params=_SC_PARAMS,
      scratch_types=(
          [pltpu.VMEM((NT1,), jnp.int32)] * NBUF1 +
          [pltpu.VMEM((NT1, D), jnp.bfloat16)] * NBUF1 +
          [pltpu.VMEM((C1, D), jnp.float32)] +
          [pltpu.SemaphoreType.DMA] * (2 * NBUF1)
      ),
  )
  return kern(news_text_flat, emb_table)


C2 = 16
NH2 = C2 * HIST
NCHUNK2 = B // C2
JMAX2 = NCHUNK2 // NW
NSLH = NH2 // ISL
HREM = NH2 - (NH2 // ISL) * ISL
def _user_gather_body(hist_hbm, uid_hbm, cand_hbm, nemb_hbm, utab_hbm,
                      hrep_hbm, uself_hbm, cemb_hbm,
                      idxhA, idxhB, idxsA, idxsB,
                      rowsA, rowsB, urowA, urowB, crowA, crowB, out_v,
                      isemA, isemB, gsemA, gsemB):
  wid = lax.axis_index("s") * NC + lax.axis_index("c")
  bufs = ((idxhA, idxsA, rowsA, urowA, crowA, isemA, gsemA),
          (idxhB, idxsB, rowsB, urowB, crowB, isemB, gsemB))

  def idx_copy(j, p):
    idxh, idxs, _, _, _, isem, _ = bufs[p]

    @pl.when(j < JMAX2)
    def _():
      g = j * NW + wid
      pltpu.make_async_copy(hist_hbm.at[pl.ds(g * NH2, NH2)], idxh,
                            isem).start()
      pltpu.make_async_copy(uid_hbm.at[pl.ds(g * C2, C2)],
                            idxs.at[pl.ds(0, C2)], isem).start()
      pltpu.make_async_copy(cand_hbm.at[pl.ds(g * C2, C2)],
                            idxs.at[pl.ds(C2, C2)], isem).start()

  def idx_wait(j, p):
    idxh, idxs, _, _, _, isem, _ = bufs[p]

    @pl.when(j < JMAX2)
    def _():
      pltpu.make_async_copy(hist_hbm.at[pl.ds(0, NH2)], idxh, isem).wait()
      pltpu.make_async_copy(uid_hbm.at[pl.ds(0, C2)],
                            idxs.at[pl.ds(0, C2)], isem).wait()
      pltpu.make_async_copy(cand_hbm.at[pl.ds(0, C2)],
                            idxs.at[pl.ds(C2, C2)], isem).wait()

  def gather_start(j, p):
    idxh, idxs, rows, urow, crow, _, gsem = bufs[p]

    @pl.when(j < JMAX2)
    def _():
      for k in range(NSLH):
        pltpu.make_async_copy(
            nemb_hbm.at[idxh.at[pl.ds(k * ISL, ISL)]],
            rows.at[pl.ds(k * ISL, ISL)], gsem).start()
      pltpu.make_async_copy(
          nemb_hbm.at[idxh.at[pl.ds(NSLH * ISL, HREM)]],
          rows.at[pl.ds(NSLH * ISL, HREM)], gsem).start()
      pltpu.make_async_copy(nemb_hbm.at[idxs.at[pl.ds(C2, C2)]], crow,
                            gsem).start()
      pltpu.make_async_copy(utab_hbm.at[idxs.at[pl.ds(0, C2)]], urow,
                            gsem).start()

  def gather_wait(j, p):
    idxh, idxs, rows, urow, crow, _, gsem = bufs[p]

    @pl.when(j < JMAX2)
    def _():
      for k in range(NSLH):
        pltpu.make_async_copy(
            nemb_hbm.at[idxh.at[pl.ds(k * ISL, ISL)]],
            rows.at[pl.ds(k * ISL, ISL)], gsem).wait()
      pltpu.make_async_copy(
          nemb_hbm.at[idxh.at[pl.ds(NSLH * ISL, HREM)]],
          rows.at[pl.ds(NSLH * ISL, HREM)], gsem).wait()
      pltpu.make_async_copy(nemb_hbm.at[idxs.at[pl.ds(C2, C2)]], crow,
                            gsem).wait()
      pltpu.make_async_copy(utab_hbm.at[idxs.at[pl.ds(0, C2)]], urow,
                            gsem).wait()

  def compute(j, p):
    _, _, rows, urow, crow, _, _ = bufs[p]

    @pl.when(j < JMAX2)
    def _():
      g = j * NW + wid

      def row_body(c, _):
        base = c * HIST
        for d in range(D // L):
          acc = rows[base, pl.ds(d * L, L)]
          for t in range(1, HIST):
            acc = acc + rows[base + t, pl.ds(d * L, L)]
          out_v[c, pl.ds(d * L, L)] = acc * (1.0 / HIST)
        return 0

      lax.fori_loop(0, C2, row_body, 0)
      pltpu.sync_copy(out_v, hrep_hbm.at[pl.ds(g * C2, C2)])
      pltpu.sync_copy(urow, uself_hbm.at[pl.ds(g * C2, C2)])
      pltpu.sync_copy(crow, cemb_hbm.at[pl.ds(g * C2, C2)])

  idx_copy(0, 0)
  idx_copy(1, 1)
  idx_wait(0, 0)
  gather_start(0, 0)

  def loop_body(jj, _):
    for p in (0, 1):
      j = 2 * jj + p
      idx_wait(j + 1, 1 - p)
      gather_start(j + 1, 1 - p)
      gather_wait(j, p)
      compute(j, p)
      idx_copy(j + 2, p)
    return 0

  lax.fori_loop(0, JMAX2 // 2, loop_body, 0)


@jax.jit
def _user_gather(hist_flat, user_ids, cand_ids, news_emb, user_table):
  mesh = plsc.VectorSubcoreMesh(core_axis_name="c", subcore_axis_name="s")
  sds = jax.ShapeDtypeStruct((B, D), jnp.float32)
  kern = pl.kernel(
      _user_gather_body,
      out_type=(sds, sds, sds),
      mesh=mesh,
      compiler_params=_SC_PARAMS,
      scratch_types=[
          pltpu.VMEM((NH2,), jnp.int32),
          pltpu.VMEM((NH2,), jnp.int32),
          pltpu.VMEM((2 * C2,), jnp.int32),
          pltpu.VMEM((2 * C2,), jnp.int32),
          pltpu.VMEM((NH2, D), jnp.float32),
          pltpu.VMEM((NH2, D), jnp.float32),
          pltpu.VMEM((C2, D), jnp.float32),
          pltpu.VMEM((C2, D), jnp.float32),
          pltpu.VMEM((C2, D), jnp.float32),
          pltpu.VMEM((C2, D), jnp.float32),
          pltpu.VMEM((C2, D), jnp.float32),
          pltpu.SemaphoreType.DMA,
          pltpu.SemaphoreType.DMA,
          pltpu.SemaphoreType.DMA,
          pltpu.SemaphoreType.DMA,
      ],
  )
  return kern(hist_flat, user_ids, cand_ids, news_emb, user_table)


def _dense_body(u_ref, h_ref, c_ref, ws_ref, bs_ref, wa_ref, ba_ref, o_ref):
  dn = (((1,), (1,)), ((), ()))
  x = lax.dot_general(u_ref[...], ws_ref[...], dn,
                      preferred_element_type=jnp.float32)
  y = lax.dot_general(h_ref[...], wa_ref[...], dn,
                      preferred_element_type=jnp.float32)
  z = jnp.tanh(x + y + (bs_ref[...] + ba_ref[...])[None, :])
  o_ref[...] = jnp.sum(z * c_ref[...], axis=1)


@jax.jit
def _dense_score(u_self, hist_rep, cand_emb, Wself, bself, Waggr, baggr):
  return pl.pallas_call(
      _dense_body,
      out_shape=jax.ShapeDtypeStruct((B,), jnp.float32),
  )(u_self, hist_rep, cand_emb, Wself, bself, Waggr, baggr)


def kernel(news_text, user_history_batch, user_ids, candidate_news_ids,
           emb_table, user_table, Wself, bself, Waggr, baggr):
  news_text_flat = jnp.reshape(news_text.astype(jnp.int32), (-1,))
  hist_flat = jnp.reshape(user_history_batch.astype(jnp.int32), (-1,))
  user_ids = user_ids.astype(jnp.int32)
  cand_ids = candidate_news_ids.astype(jnp.int32)

  news_emb = _news_pool(news_text_flat, emb_table.astype(jnp.bfloat16))
  hist_rep, u_self, cand_emb = _user_gather(
      hist_flat, user_ids, cand_ids, news_emb, user_table)
  return _dense_score(u_self, hist_rep, cand_emb, Wself, bself, Waggr,
                      baggr)


# speedup vs baseline: 1.0119x; 1.0119x over previous
import jax
import jax.numpy as jnp
from jax import lax
from jax.experimental import pallas as pl
from jax.experimental.pallas import tpu as pltpu
from jax.experimental.pallas import tpu_sc as plsc

NUM_NEWS = 100000
NUM_USERS = 1000000
MAX_LEN = 20
D = 64
B = 4096
HIST = 50

NC = 2
NS = 16
NW = NC * NS
L = 16

_SC_PARAMS = pltpu.CompilerParams(needs_layout_passes=False,
                                  use_tc_tiling_on_sc=False)


C1 = 16
NT1 = C1 * MAX_LEN
NCHUNK1 = NUM_NEWS // C1
JMAX1 = (NCHUNK1 + NW - 1) // NW
ISL = 128
SL1 = ((0, 128), (128, 128), (256, 64))
NBUF1 = 4


def _news_pool_body(text_hbm, table_hbm, out_hbm,
                    idx0, idx1, idx2, idx3, rows0, rows1, rows2, rows3,
                    out_v, inv_v, isem0, isem1, isem2, isem3,
                    gsem0, gsem1, gsem2, gsem3):
  wid = lax.axis_index("s") * NC + lax.axis_index("c")
  lanes = lax.iota(jnp.int32, L)
  bufs = ((idx0, rows0, isem0, gsem0), (idx1, rows1, isem1, gsem1),
          (idx2, rows2, isem2, gsem2), (idx3, rows3, isem3, gsem3))

  def g_of(j):
    return j * NW + wid

  def idx_copy(j, p):
    idx, _, isem, _ = bufs[p]
    g = g_of(j)

    @pl.when(g < NCHUNK1)
    def _():
      pltpu.make_async_copy(text_hbm.at[pl.ds(g * NT1, NT1)], idx,
                            isem).start()

  def idx_wait(j, p):
    idx, _, isem, _ = bufs[p]

    @pl.when(g_of(j) < NCHUNK1)
    def _():
      pltpu.make_async_copy(text_hbm.at[pl.ds(0, NT1)], idx, isem).wait()

  def gather_start(j, p):
    idx, rows, _, gsem = bufs[p]

    @pl.when(g_of(j) < NCHUNK1)
    def _():
      for o, n in SL1:
        pltpu.make_async_copy(
            table_hbm.at[idx.at[pl.ds(o, n)]],
            rows.at[pl.ds(o, n)], gsem).start()

  def gather_wait(j, p):
    idx, rows, _, gsem = bufs[p]

    @pl.when(g_of(j) < NCHUNK1)
    def _():
      for o, n in SL1:
        pltpu.make_async_copy(
            table_hbm.at[idx.at[pl.ds(o, n)]],
            rows.at[pl.ds(o, n)], gsem).wait()

  def compute(j, p):
    idx, rows, _, _ = bufs[p]
    g = g_of(j)

    @pl.when(g < NCHUNK1)
    def _():
      ones = jnp.ones((L,), jnp.float32)
      zeros = jnp.zeros((L,), jnp.float32)
      cnt = zeros
      for t in range(MAX_LEN):
        tok = plsc.load_gather(idx, [lanes * MAX_LEN + t])
        cnt = cnt + jnp.where(tok != 0, ones, zeros)
      inv_v[...] = ones / jnp.maximum(cnt, 1e-9)

      def row_body(c4, _):
        mask_hi = jnp.full((L,), 0xFFFF0000, jnp.uint32)
        for i in range(4):
          c = c4 * 4 + i
          base = c * MAX_LEN
          s = plsc.load_gather(inv_v, [jnp.full((L,), c, jnp.int32)])
          cvec = jnp.full((L,), c, jnp.int32)
          for h in range(D // 32):
            acc_lo = jnp.zeros((L,), jnp.float32)
            acc_hi = jnp.zeros((L,), jnp.float32)
            for t in range(MAX_LEN):
              w = plsc.bitcast(rows[base + t, pl.ds(h * 32, 32)],
                               jnp.uint32)
              acc_lo = acc_lo + plsc.bitcast(w << 16, jnp.float32)
              acc_hi = acc_hi + plsc.bitcast(w & mask_hi, jnp.float32)
            plsc.store_scatter(out_v, [cvec, h * 32 + 2 * lanes],
                               acc_lo * s)
            plsc.store_scatter(out_v, [cvec, h * 32 + 2 * lanes + 1],
                               acc_hi * s)
        return 0

      lax.fori_loop(0, C1 // 4, row_body, 0)
      pltpu.sync_copy(out_v, out_hbm.at[pl.ds(g * C1, C1)])

  for p in range(NBUF1):
    idx_copy(p, p)
  idx_wait(0, 0)
  gather_start(0, 0)
  idx_wait(1, 1)
  gather_start(1, 1)

  def loop_body(jj, _):
    for p in range(NBUF1):
      j = NBUF1 * jj + p
      p2 = (p + 2) % NBUF1
      idx_wait(j + 2, p2)
      gather_start(j + 2, p2)
      gather_wait(j, p)
      compute(j, p)
      idx_copy(j + NBUF1, p)
    return 0

  lax.fori_loop(0, JMAX1 // NBUF1, loop_body, 0)


@jax.jit
def _news_pool(news_text_flat, emb_table):
  mesh = plsc.VectorSubcoreMesh(core_axis_name="c", subcore_axis_name="s")
  kern = pl.kernel(
      _news_pool_body,
      out_type=jax.ShapeDtypeStruct((NUM_NEWS, D), jnp.float32),
      mesh=mesh,
      compiler_params=_SC_PARAMS,
      scratch_types=(
          [pltpu.VMEM((NT1,), jnp.int32)] * NBUF1 +
          [pltpu.VMEM((NT1, D), jnp.bfloat16)] * NBUF1 +
          [pltpu.VMEM((C1, D), jnp.float32)] +
          [pltpu.VMEM((L,), jnp.float32)] +
          [pltpu.SemaphoreType.DMA] * (2 * NBUF1)
      ),
  )
  return kern(news_text_flat, emb_table)


C2 = 16
NH2 = C2 * HIST
NCHUNK2 = B // C2
JMAX2 = NCHUNK2 // NW
NSLH = NH2 // ISL
HREM = NH2 - (NH2 // ISL) * ISL
def _user_gather_body(hist_hbm, uid_hbm, cand_hbm, nemb_hbm, utab_hbm,
                      hrep_hbm, uself_hbm, cemb_hbm,
                      idxhA, idxhB, idxsA, idxsB,
                      rowsA, rowsB, urowA, urowB, crowA, crowB, out_v,
                      isemA, isemB, gsemA, gsemB):
  wid = lax.axis_index("s") * NC + lax.axis_index("c")
  bufs = ((idxhA, idxsA, rowsA, urowA, crowA, isemA, gsemA),
          (idxhB, idxsB, rowsB, urowB, crowB, isemB, gsemB))

  def idx_copy(j, p):
    idxh, idxs, _, _, _, isem, _ = bufs[p]

    @pl.when(j < JMAX2)
    def _():
      g = j * NW + wid
      pltpu.make_async_copy(hist_hbm.at[pl.ds(g * NH2, NH2)], idxh,
                            isem).start()
      pltpu.make_async_copy(uid_hbm.at[pl.ds(g * C2, C2)],
                            idxs.at[pl.ds(0, C2)], isem).start()
      pltpu.make_async_copy(cand_hbm.at[pl.ds(g * C2, C2)],
                            idxs.at[pl.ds(C2, C2)], isem).start()

  def idx_wait(j, p):
    idxh, idxs, _, _, _, isem, _ = bufs[p]

    @pl.when(j < JMAX2)
    def _():
      pltpu.make_async_copy(hist_hbm.at[pl.ds(0, NH2)], idxh, isem).wait()
      pltpu.make_async_copy(uid_hbm.at[pl.ds(0, C2)],
                            idxs.at[pl.ds(0, C2)], isem).wait()
      pltpu.make_async_copy(cand_hbm.at[pl.ds(0, C2)],
                            idxs.at[pl.ds(C2, C2)], isem).wait()

  def gather_start(j, p):
    idxh, idxs, rows, urow, crow, _, gsem = bufs[p]

    @pl.when(j < JMAX2)
    def _():
      for k in range(NSLH):
        pltpu.make_async_copy(
            nemb_hbm.at[idxh.at[pl.ds(k * ISL, ISL)]],
            rows.at[pl.ds(k * ISL, ISL)], gsem).start()
      pltpu.make_async_copy(
          nemb_hbm.at[idxh.at[pl.ds(NSLH * ISL, HREM)]],
          rows.at[pl.ds(NSLH * ISL, HREM)], gsem).start()
      pltpu.make_async_copy(nemb_hbm.at[idxs.at[pl.ds(C2, C2)]], crow,
                            gsem).start()
      pltpu.make_async_copy(utab_hbm.at[idxs.at[pl.ds(0, C2)]], urow,
                            gsem).start()

  def gather_wait(j, p):
    idxh, idxs, rows, urow, crow, _, gsem = bufs[p]

    @pl.when(j < JMAX2)
    def _():
      for k in range(NSLH):
        pltpu.make_async_copy(
            nemb_hbm.at[idxh.at[pl.ds(k * ISL, ISL)]],
            rows.at[pl.ds(k * ISL, ISL)], gsem).wait()
      pltpu.make_async_copy(
          nemb_hbm.at[idxh.at[pl.ds(NSLH * ISL, HREM)]],
          rows.at[pl.ds(NSLH * ISL, HREM)], gsem).wait()
      pltpu.make_async_copy(nemb_hbm.at[idxs.at[pl.ds(C2, C2)]], crow,
                            gsem).wait()
      pltpu.make_async_copy(utab_hbm.at[idxs.at[pl.ds(0, C2)]], urow,
                            gsem).wait()

  def compute(j, p):
    _, _, rows, urow, crow, _, _ = bufs[p]

    @pl.when(j < JMAX2)
    def _():
      g = j * NW + wid

      def row_body(c, _):
        base = c * HIST
        for d in range(D // L):
          acc = rows[base, pl.ds(d * L, L)]
          for t in range(1, HIST):
            acc = acc + rows[base + t, pl.ds(d * L, L)]
          out_v[c, pl.ds(d * L, L)] = acc * (1.0 / HIST)
        return 0

      lax.fori_loop(0, C2, row_body, 0)
      pltpu.sync_copy(out_v, hrep_hbm.at[pl.ds(g * C2, C2)])
      pltpu.sync_copy(urow, uself_hbm.at[pl.ds(g * C2, C2)])
      pltpu.sync_copy(crow, cemb_hbm.at[pl.ds(g * C2, C2)])

  idx_copy(0, 0)
  idx_copy(1, 1)
  idx_wait(0, 0)
  gather_start(0, 0)

  def loop_body(jj, _):
    for p in (0, 1):
      j = 2 * jj + p
      idx_wait(j + 1, 1 - p)
      gather_start(j + 1, 1 - p)
      gather_wait(j, p)
      compute(j, p)
      idx_copy(j + 2, p)
    return 0

  lax.fori_loop(0, JMAX2 // 2, loop_body, 0)


@jax.jit
def _user_gather(hist_flat, user_ids, cand_ids, news_emb, user_table):
  mesh = plsc.VectorSubcoreMesh(core_axis_name="c", subcore_axis_name="s")
  sds = jax.ShapeDtypeStruct((B, D), jnp.float32)
  kern = pl.kernel(
      _user_gather_body,
      out_type=(sds, sds, sds),
      mesh=mesh,
      compiler_params=_SC_PARAMS,
      scratch_types=[
          pltpu.VMEM((NH2,), jnp.int32),
          pltpu.VMEM((NH2,), jnp.int32),
          pltpu.VMEM((2 * C2,), jnp.int32),
          pltpu.VMEM((2 * C2,), jnp.int32),
          pltpu.VMEM((NH2, D), jnp.float32),
          pltpu.VMEM((NH2, D), jnp.float32),
          pltpu.VMEM((C2, D), jnp.float32),
          pltpu.VMEM((C2, D), jnp.float32),
          pltpu.VMEM((C2, D), jnp.float32),
          pltpu.VMEM((C2, D), jnp.float32),
          pltpu.VMEM((C2, D), jnp.float32),
          pltpu.SemaphoreType.DMA,
          pltpu.SemaphoreType.DMA,
          pltpu.SemaphoreType.DMA,
          pltpu.SemaphoreType.DMA,
      ],
  )
  return kern(hist_flat, user_ids, cand_ids, news_emb, user_table)


def _dense_body(u_ref, h_ref, c_ref, ws_ref, bs_ref, wa_ref, ba_ref, o_ref):
  dn = (((1,), (1,)), ((), ()))
  x = lax.dot_general(u_ref[...], ws_ref[...], dn,
                      preferred_element_type=jnp.float32)
  y = lax.dot_general(h_ref[...], wa_ref[...], dn,
                      preferred_element_type=jnp.float32)
  z = jnp.tanh(x + y + (bs_ref[...] + ba_ref[...])[None, :])
  o_ref[...] = jnp.sum(z * c_ref[...], axis=1)


@jax.jit
def _dense_score(u_self, hist_rep, cand_emb, Wself, bself, Waggr, baggr):
  return pl.pallas_call(
      _dense_body,
      out_shape=jax.ShapeDtypeStruct((B,), jnp.float32),
  )(u_self, hist_rep, cand_emb, Wself, bself, Waggr, baggr)


def kernel(news_text, user_history_batch, user_ids, candidate_news_ids,
           emb_table, user_table, Wself, bself, Waggr, baggr):
  news_text_flat = jnp.reshape(news_text.astype(jnp.int32), (-1,))
  hist_flat = jnp.reshape(user_history_batch.astype(jnp.int32), (-1,))
  user_ids = user_ids.astype(jnp.int32)
  cand_ids = candidate_news_ids.astype(jnp.int32)

  news_emb = _news_pool(news_text_flat, emb_table.astype(jnp.bfloat16))
  hist_rep, u_self, cand_emb = _user_gather(
      hist_flat, user_ids, cand_ids, news_emb, user_table)
  return _dense_score(u_self, hist_rep, cand_emb, Wself, bself, Waggr,
                      baggr)


# speedup vs baseline: 1.0210x; 1.0090x over previous
import jax
import jax.numpy as jnp
from jax import lax
from jax.experimental import pallas as pl
from jax.experimental.pallas import tpu as pltpu
from jax.experimental.pallas import tpu_sc as plsc

NUM_NEWS = 100000
NUM_USERS = 1000000
MAX_LEN = 20
D = 64
B = 4096
HIST = 50

NC = 2
NS = 16
NW = NC * NS
L = 16

_SC_PARAMS = pltpu.CompilerParams(needs_layout_passes=False,
                                  use_tc_tiling_on_sc=False)


C1 = 32
NT1 = C1 * MAX_LEN
NCHUNK1 = NUM_NEWS // C1
ISL = 128
SL1 = tuple((k * ISL, ISL) for k in range(NT1 // ISL))
NBUF1 = 3
JMAX1 = 99


def _news_pool_body(text_hbm, table_hbm, out_hbm,
                    idx0, idx1, idx2, rows0, rows1, rows2,
                    out0, out1, out2, inv_v,
                    isem0, isem1, isem2, gsem0, gsem1, gsem2,
                    osem0, osem1, osem2):
  wid = lax.axis_index("s") * NC + lax.axis_index("c")
  lanes = lax.iota(jnp.int32, L)
  bufs = ((idx0, rows0, out0, isem0, gsem0, osem0),
          (idx1, rows1, out1, isem1, gsem1, osem1),
          (idx2, rows2, out2, isem2, gsem2, osem2))

  def g_of(j):
    return j * NW + wid

  def idx_copy(j, p):
    idx, _, _, isem, _, _ = bufs[p]
    g = g_of(j)

    @pl.when(g < NCHUNK1)
    def _():
      pltpu.make_async_copy(text_hbm.at[pl.ds(g * NT1, NT1)], idx,
                            isem).start()

  def idx_wait(j, p):
    idx, _, _, isem, _, _ = bufs[p]

    @pl.when(g_of(j) < NCHUNK1)
    def _():
      pltpu.make_async_copy(text_hbm.at[pl.ds(0, NT1)], idx, isem).wait()

  def gather_start(j, p):
    idx, rows, _, _, gsem, _ = bufs[p]

    @pl.when(g_of(j) < NCHUNK1)
    def _():
      for o, n in SL1:
        pltpu.make_async_copy(
            table_hbm.at[idx.at[pl.ds(o, n)]],
            rows.at[pl.ds(o, n)], gsem).start()

  def gather_wait(j, p):
    idx, rows, _, _, gsem, _ = bufs[p]

    @pl.when(g_of(j) < NCHUNK1)
    def _():
      for o, n in SL1:
        pltpu.make_async_copy(
            table_hbm.at[idx.at[pl.ds(o, n)]],
            rows.at[pl.ds(o, n)], gsem).wait()

  def compute(j, p):
    idx, rows, out_v, _, _, osem = bufs[p]
    g = g_of(j)

    @pl.when((j >= NBUF1) & (g < NCHUNK1))
    def _():
      pltpu.make_async_copy(out_v, out_hbm.at[pl.ds(0, C1)], osem).wait()

    @pl.when(g < NCHUNK1)
    def _():
      ones = jnp.ones((L,), jnp.float32)
      zeros = jnp.zeros((L,), jnp.float32)
      for half in range(C1 // L):
        cnt = zeros
        for t in range(MAX_LEN):
          tok = plsc.load_gather(idx, [(lanes + half * L) * MAX_LEN + t])
          cnt = cnt + jnp.where(tok != 0, ones, zeros)
        inv_v[pl.ds(half * L, L)] = ones / jnp.maximum(cnt, 1e-9)

      def row_body(c4, _):
        mask_hi = jnp.full((L,), 0xFFFF0000, jnp.uint32)
        for i in range(4):
          c = c4 * 4 + i
          base = c * MAX_LEN
          s = plsc.load_gather(inv_v, [jnp.full((L,), c, jnp.int32)])
          cvec = jnp.full((L,), c, jnp.int32)
          for h in range(D // 32):
            acc_lo = jnp.zeros((L,), jnp.float32)
            acc_hi = jnp.zeros((L,), jnp.float32)
            for t in range(MAX_LEN):
              w = plsc.bitcast(rows[base + t, pl.ds(h * 32, 32)],
                               jnp.uint32)
              acc_lo = acc_lo + plsc.bitcast(w << 16, jnp.float32)
              acc_hi = acc_hi + plsc.bitcast(w & mask_hi, jnp.float32)
            plsc.store_scatter(out_v, [cvec, h * 32 + 2 * lanes],
                               acc_lo * s)
            plsc.store_scatter(out_v, [cvec, h * 32 + 2 * lanes + 1],
                               acc_hi * s)
        return 0

      lax.fori_loop(0, C1 // 4, row_body, 0)
      pltpu.make_async_copy(out_v, out_hbm.at[pl.ds(g * C1, C1)],
                            osem).start()

  for p in range(NBUF1):
    idx_copy(p, p)
  idx_wait(0, 0)
  gather_start(0, 0)
  idx_wait(1, 1)
  gather_start(1, 1)

  def loop_body(jj, _):
    for p in range(NBUF1):
      j = NBUF1 * jj + p
      p2 = (p + 2) % NBUF1
      idx_wait(j + 2, p2)
      gather_start(j + 2, p2)
      gather_wait(j, p)
      compute(j, p)
      idx_copy(j + NBUF1, p)
    return 0

  lax.fori_loop(0, JMAX1 // NBUF1, loop_body, 0)

  for p in range(NBUF1):
    out_v = bufs[p][2]
    osem = bufs[p][5]
    for jc in (JMAX1 - NBUF1 + p, JMAX1 - 2 * NBUF1 + p):

      @pl.when((g_of(jc) < NCHUNK1) & (g_of(jc + NBUF1) >= NCHUNK1))
      def _():
        pltpu.make_async_copy(out_v, out_hbm.at[pl.ds(0, C1)],
                              osem).wait()


@jax.jit
def _news_pool(news_text_flat, emb_table):
  mesh = plsc.VectorSubcoreMesh(core_axis_name="c", subcore_axis_name="s")
  kern = pl.kernel(
      _news_pool_body,
      out_type=jax.ShapeDtypeStruct((NUM_NEWS, D), jnp.float32),
      mesh=mesh,
      compiler_params=_SC_PARAMS,
      scratch_types=(
          [pltpu.VMEM((NT1,), jnp.int32)] * NBUF1 +
          [pltpu.VMEM((NT1, D), jnp.bfloat16)] * NBUF1 +
          [pltpu.VMEM((C1, D), jnp.float32)] * NBUF1 +
          [pltpu.VMEM((C1,), jnp.float32)] +
          [pltpu.SemaphoreType.DMA] * (3 * NBUF1)
      ),
  )
  return kern(news_text_flat, emb_table)


C2 = 16
NH2 = C2 * HIST
NCHUNK2 = B // C2
JMAX2 = NCHUNK2 // NW
NSLH = NH2 // ISL
HREM = NH2 - (NH2 // ISL) * ISL
def _user_gather_body(hist_hbm, uid_hbm, cand_hbm, nemb_hbm, utab_hbm,
                      hrep_hbm, uself_hbm, cemb_hbm,
                      idxhA, idxhB, idxsA, idxsB,
                      rowsA, rowsB, urowA, urowB, crowA, crowB, out_v,
                      isemA, isemB, gsemA, gsemB):
  wid = lax.axis_index("s") * NC + lax.axis_index("c")
  bufs = ((idxhA, idxsA, rowsA, urowA, crowA, isemA, gsemA),
          (idxhB, idxsB, rowsB, urowB, crowB, isemB, gsemB))

  def idx_copy(j, p):
    idxh, idxs, _, _, _, isem, _ = bufs[p]

    @pl.when(j < JMAX2)
    def _():
      g = j * NW + wid
      pltpu.make_async_copy(hist_hbm.at[pl.ds(g * NH2, NH2)], idxh,
                            isem).start()
      pltpu.make_async_copy(uid_hbm.at[pl.ds(g * C2, C2)],
                            idxs.at[pl.ds(0, C2)], isem).start()
      pltpu.make_async_copy(cand_hbm.at[pl.ds(g * C2, C2)],
                            idxs.at[pl.ds(C2, C2)], isem).start()

  def idx_wait(j, p):
    idxh, idxs, _, _, _, isem, _ = bufs[p]

    @pl.when(j < JMAX2)
    def _():
      pltpu.make_async_copy(hist_hbm.at[pl.ds(0, NH2)], idxh, isem).wait()
      pltpu.make_async_copy(uid_hbm.at[pl.ds(0, C2)],
                            idxs.at[pl.ds(0, C2)], isem).wait()
      pltpu.make_async_copy(cand_hbm.at[pl.ds(0, C2)],
                            idxs.at[pl.ds(C2, C2)], isem).wait()

  def gather_start(j, p):
    idxh, idxs, rows, urow, crow, _, gsem = bufs[p]

    @pl.when(j < JMAX2)
    def _():
      for k in range(NSLH):
        pltpu.make_async_copy(
            nemb_hbm.at[idxh.at[pl.ds(k * ISL, ISL)]],
            rows.at[pl.ds(k * ISL, ISL)], gsem).start()
      pltpu.make_async_copy(
          nemb_hbm.at[idxh.at[pl.ds(NSLH * ISL, HREM)]],
          rows.at[pl.ds(NSLH * ISL, HREM)], gsem).start()
      pltpu.make_async_copy(nemb_hbm.at[idxs.at[pl.ds(C2, C2)]], crow,
                            gsem).start()
      pltpu.make_async_copy(utab_hbm.at[idxs.at[pl.ds(0, C2)]], urow,
                            gsem).start()

  def gather_wait(j, p):
    idxh, idxs, rows, urow, crow, _, gsem = bufs[p]

    @pl.when(j < JMAX2)
    def _():
      for k in range(NSLH):
        pltpu.make_async_copy(
            nemb_hbm.at[idxh.at[pl.ds(k * ISL, ISL)]],
            rows.at[pl.ds(k * ISL, ISL)], gsem).wait()
      pltpu.make_async_copy(
          nemb_hbm.at[idxh.at[pl.ds(NSLH * ISL, HREM)]],
          rows.at[pl.ds(NSLH * ISL, HREM)], gsem).wait()
      pltpu.make_async_copy(nemb_hbm.at[idxs.at[pl.ds(C2, C2)]], crow,
                            gsem).wait()
      pltpu.make_async_copy(utab_hbm.at[idxs.at[pl.ds(0, C2)]], urow,
                            gsem).wait()

  def compute(j, p):
    _, _, rows, urow, crow, _, _ = bufs[p]

    @pl.when(j < JMAX2)
    def _():
      g = j * NW + wid

      def row_body(c, _):
        base = c * HIST
        for d in range(D // L):
          acc = rows[base, pl.ds(d * L, L)]
          for t in range(1, HIST):
            acc = acc + rows[base + t, pl.ds(d * L, L)]
          out_v[c, pl.ds(d * L, L)] = acc * (1.0 / HIST)
        return 0

      lax.fori_loop(0, C2, row_body, 0)
      pltpu.sync_copy(out_v, hrep_hbm.at[pl.ds(g * C2, C2)])
      pltpu.sync_copy(urow, uself_hbm.at[pl.ds(g * C2, C2)])
      pltpu.sync_copy(crow, cemb_hbm.at[pl.ds(g * C2, C2)])

  idx_copy(0, 0)
  idx_copy(1, 1)
  idx_wait(0, 0)
  gather_start(0, 0)

  def loop_body(jj, _):
    for p in (0, 1):
      j = 2 * jj + p
      idx_wait(j + 1, 1 - p)
      gather_start(j + 1, 1 - p)
      gather_wait(j, p)
      compute(j, p)
      idx_copy(j + 2, p)
    return 0

  lax.fori_loop(0, JMAX2 // 2, loop_body, 0)


@jax.jit
def _user_gather(hist_flat, user_ids, cand_ids, news_emb, user_table):
  mesh = plsc.VectorSubcoreMesh(core_axis_name="c", subcore_axis_name="s")
  sds = jax.ShapeDtypeStruct((B, D), jnp.float32)
  kern = pl.kernel(
      _user_gather_body,
      out_type=(sds, sds, sds),
      mesh=mesh,
      compiler_params=_SC_PARAMS,
      scratch_types=[
          pltpu.VMEM((NH2,), jnp.int32),
          pltpu.VMEM((NH2,), jnp.int32),
          pltpu.VMEM((2 * C2,), jnp.int32),
          pltpu.VMEM((2 * C2,), jnp.int32),
          pltpu.VMEM((NH2, D), jnp.float32),
          pltpu.VMEM((NH2, D), jnp.float32),
          pltpu.VMEM((C2, D), jnp.float32),
          pltpu.VMEM((C2, D), jnp.float32),
          pltpu.VMEM((C2, D), jnp.float32),
          pltpu.VMEM((C2, D), jnp.float32),
          pltpu.VMEM((C2, D), jnp.float32),
          pltpu.SemaphoreType.DMA,
          pltpu.SemaphoreType.DMA,
          pltpu.SemaphoreType.DMA,
          pltpu.SemaphoreType.DMA,
      ],
  )
  return kern(hist_flat, user_ids, cand_ids, news_emb, user_table)


def _dense_body(u_ref, h_ref, c_ref, ws_ref, bs_ref, wa_ref, ba_ref, o_ref):
  dn = (((1,), (1,)), ((), ()))
  x = lax.dot_general(u_ref[...], ws_ref[...], dn,
                      preferred_element_type=jnp.float32)
  y = lax.dot_general(h_ref[...], wa_ref[...], dn,
                      preferred_element_type=jnp.float32)
  z = jnp.tanh(x + y + (bs_ref[...] + ba_ref[...])[None, :])
  o_ref[...] = jnp.sum(z * c_ref[...], axis=1)


@jax.jit
def _dense_score(u_self, hist_rep, cand_emb, Wself, bself, Waggr, baggr):
  return pl.pallas_call(
      _dense_body,
      out_shape=jax.ShapeDtypeStruct((B,), jnp.float32),
  )(u_self, hist_rep, cand_emb, Wself, bself, Waggr, baggr)


def kernel(news_text, user_history_batch, user_ids, candidate_news_ids,
           emb_table, user_table, Wself, bself, Waggr, baggr):
  news_text_flat = jnp.reshape(news_text.astype(jnp.int32), (-1,))
  hist_flat = jnp.reshape(user_history_batch.astype(jnp.int32), (-1,))
  user_ids = user_ids.astype(jnp.int32)
  cand_ids = candidate_news_ids.astype(jnp.int32)

  news_emb = _news_pool(news_text_flat, emb_table.astype(jnp.bfloat16))
  hist_rep, u_self, cand_emb = _user_gather(
      hist_flat, user_ids, cand_ids, news_emb, user_table)
  return _dense_score(u_self, hist_rep, cand_emb, Wself, bself, Waggr,
                      baggr)
